# baseline (speedup 1.0000x reference)
import functools

import jax
import jax.numpy as jnp
from jax import lax
from jax.experimental import pallas as pl
from jax.experimental.pallas import tpu as pltpu
from jax.experimental.pallas import tpu_sc as plsc

_VOCAB = 1000000
_EMBED_DIM = 64
_BATCH = 4096
_SEQ_LEN = 200
_B = _BATCH * _SEQ_LEN

_NC = 2
_NS = 16
_NW = _NC * _NS
_B_PER_W = _B // _NW
_CHUNK = 128
_NCHUNKS = _B_PER_W // _CHUNK


def _gather_body(ids_hbm, table_hbm, out_hbm, idx_v, rows_v, sem):
    wid = lax.axis_index("s") * _NC + lax.axis_index("c")
    base = wid * _B_PER_W

    def body(i, carry):
        off = base + i * _CHUNK
        pltpu.sync_copy(ids_hbm.at[pl.ds(off, _CHUNK)], idx_v)
        pltpu.async_copy(table_hbm.at[idx_v], rows_v, sem).wait()
        pltpu.sync_copy(rows_v, out_hbm.at[pl.ds(off, _CHUNK)])
        return carry

    lax.fori_loop(0, _NCHUNKS, body, 0)


def kernel(input_ids, table):
    ids = input_ids.reshape(_B).astype(jnp.int32)

    mesh = plsc.VectorSubcoreMesh(core_axis_name="c", subcore_axis_name="s")
    gather = functools.partial(
        pl.kernel,
        mesh=mesh,
        out_type=jax.ShapeDtypeStruct((_B, _EMBED_DIM), jnp.float32),
        scratch_types=[
            pltpu.VMEM((_CHUNK,), jnp.int32),
            pltpu.VMEM((_CHUNK, _EMBED_DIM), jnp.float32),
            pltpu.SemaphoreType.DMA,
        ],
        compiler_params=pltpu.CompilerParams(use_tc_tiling_on_sc=False),
    )(_gather_body)

    out = gather(ids, table)
    return out.reshape(_BATCH, _SEQ_LEN, _EMBED_DIM)


# speedup vs baseline: 1.1824x; 1.1824x over previous
import functools

import jax
import jax.numpy as jnp
from jax import lax
from jax.experimental import pallas as pl
from jax.experimental.pallas import tpu as pltpu
from jax.experimental.pallas import tpu_sc as plsc

_VOCAB = 1000000
_EMBED_DIM = 64
_BATCH = 4096
_SEQ_LEN = 200
_B = _BATCH * _SEQ_LEN

_NC = 2
_NS = 16
_NW = _NC * _NS
_B_PER_W = _B // _NW
_CHUNK = 128
_NCHUNKS = _B_PER_W // _CHUNK
_NBUF = 4
_LOOKAHEAD = 2


def _gather_pipeline(ids_hbm, table_hbm, out_hbm, idx_v, rows_v, gsem, ssem):
    wid = lax.axis_index("s") * _NC + lax.axis_index("c")
    base = wid * _B_PER_W

    pltpu.sync_copy(ids_hbm.at[wid], idx_v)

    def start_gather(j, b):
        pltpu.async_copy(table_hbm.at[idx_v.at[j]], rows_v.at[b], gsem.at[b])

    def wait_gather(b):
        pltpu.make_async_copy(
            table_hbm.at[idx_v.at[0]], rows_v.at[b], gsem.at[b]).wait()

    def start_store(i, b):
        pltpu.async_copy(
            rows_v.at[b], out_hbm.at[pl.ds(base + i * _CHUNK, _CHUNK)],
            ssem.at[b])

    def wait_store(b):
        pltpu.make_async_copy(
            rows_v.at[b], out_hbm.at[pl.ds(base, _CHUNK)], ssem.at[b]).wait()

    for i in range(_LOOKAHEAD):
        start_gather(i, i % _NBUF)
    for i in range(_LOOKAHEAD):
        b = i % _NBUF
        wait_gather(b)
        start_store(i, b)
        start_gather(i + _LOOKAHEAD, (i + _LOOKAHEAD) % _NBUF)

    n_main = _NCHUNKS - 2 * _LOOKAHEAD
    def main_body(g, carry):
        for u in range(_NBUF):
            i = _LOOKAHEAD + g * _NBUF + u
            b = (_LOOKAHEAD + u) % _NBUF
            bn = u
            wait_gather(b)
            start_store(i, b)
            wait_store(bn)
            start_gather(i + _LOOKAHEAD, bn)
        return carry
    lax.fori_loop(0, n_main // _NBUF, main_body, 0)

    for i in range(_NCHUNKS - _LOOKAHEAD, _NCHUNKS):
        b = i % _NBUF
        wait_gather(b)
        start_store(i, b)
    for b in range(_NBUF):
        wait_store(b)


def kernel(input_ids, table):
    ids = input_ids.reshape(_NW, _NCHUNKS, _CHUNK).astype(jnp.int32)

    mesh = plsc.VectorSubcoreMesh(core_axis_name="c", subcore_axis_name="s")
    gather = functools.partial(
        pl.kernel,
        mesh=mesh,
        out_type=jax.ShapeDtypeStruct((_B, _EMBED_DIM), jnp.float32),
        scratch_types=[
            pltpu.VMEM((_NCHUNKS, _CHUNK), jnp.int32),
            pltpu.VMEM((_NBUF, _CHUNK, _EMBED_DIM), jnp.float32),
            pltpu.SemaphoreType.DMA((_NBUF,)),
            pltpu.SemaphoreType.DMA((_NBUF,)),
        ],
        compiler_params=pltpu.CompilerParams(use_tc_tiling_on_sc=False),
    )(_gather_pipeline)

    out = gather(ids, table)
    return out.reshape(_BATCH, _SEQ_LEN, _EMBED_DIM)
